# Initial kernel scaffold; baseline (speedup 1.0000x reference)
import jax
import jax.numpy as jnp
from jax.experimental import pallas as pl


def kernel(x, W_router, w_gate, w_up, w_down):
    raise NotImplementedError("write your pallas kernel here")


# speedup vs baseline: 1.4409x; 1.4409x over previous
import functools

import jax
import jax.numpy as jnp
from jax.experimental import pallas as pl

T = 2048
D = 2048
E = 16
K = 8
F = 768

BT_R = 512
BT = 1024


def _router_body(x_ref, wr_ref, comb_ref):
    logits = jnp.dot(x_ref[...], wr_ref[...], preferred_element_type=jnp.float32)
    p = jax.nn.softmax(logits, axis=-1)
    pw = p
    sel = jnp.zeros_like(p, dtype=jnp.bool_)
    col = jax.lax.broadcasted_iota(jnp.int32, p.shape, 1)
    for _ in range(K):
        idx = jnp.argmax(pw, axis=-1)
        oh = col == idx[:, None]
        sel = jnp.logical_or(sel, oh)
        pw = jnp.where(oh, -jnp.inf, pw)
    wsel = jnp.where(sel, p, 0.0)
    comb_ref[...] = wsel / jnp.sum(wsel, axis=-1, keepdims=True)


def _ffn_body(x_ref, wg_ref, wu_ref, wd_ref, comb_ref, out_ref):
    e = pl.program_id(1)
    xb = x_ref[...]
    g = jnp.dot(xb, wg_ref[0], preferred_element_type=jnp.float32)
    u = jnp.dot(xb, wu_ref[0], preferred_element_type=jnp.float32)
    h = g * jax.nn.sigmoid(g) * u
    lane = jax.lax.broadcasted_iota(jnp.int32, (1, E), 1)
    w = jnp.sum(jnp.where(lane == e, comb_ref[...], 0.0), axis=1, keepdims=True)
    hs = (h * w).astype(jnp.bfloat16)
    y = jnp.dot(hs, wd_ref[0], preferred_element_type=jnp.float32)

    @pl.when(e == 0)
    def _():
        out_ref[...] = y

    @pl.when(e > 0)
    def _():
        out_ref[...] += y


@functools.partial(jax.jit, static_argnames=())
def kernel(x, W_router, w_gate, w_up, w_down):
    combine = pl.pallas_call(
        _router_body,
        grid=(T // BT_R,),
        in_specs=[
            pl.BlockSpec((BT_R, D), lambda t: (t, 0)),
            pl.BlockSpec((D, E), lambda t: (0, 0)),
        ],
        out_specs=pl.BlockSpec((BT_R, E), lambda t: (t, 0)),
        out_shape=jax.ShapeDtypeStruct((T, E), jnp.float32),
    )(x, W_router)

    xb = x.astype(jnp.bfloat16)
    wg = w_gate.astype(jnp.bfloat16)
    wu = w_up.astype(jnp.bfloat16)
    wd = w_down.astype(jnp.bfloat16)

    out = pl.pallas_call(
        _ffn_body,
        grid=(T // BT, E),
        in_specs=[
            pl.BlockSpec((BT, D), lambda t, e: (t, 0)),
            pl.BlockSpec((1, D, F), lambda t, e: (e, 0, 0)),
            pl.BlockSpec((1, D, F), lambda t, e: (e, 0, 0)),
            pl.BlockSpec((1, F, D), lambda t, e: (e, 0, 0)),
            pl.BlockSpec((BT, E), lambda t, e: (t, 0)),
        ],
        out_specs=pl.BlockSpec((BT, D), lambda t, e: (t, 0)),
        out_shape=jax.ShapeDtypeStruct((T, D), jnp.float32),
    )(xb, wg, wu, wd, combine)
    return out
